# baseline (speedup 1.0000x reference)
import functools

import jax
import jax.numpy as jnp
from jax import lax
from jax.experimental import pallas as pl
from jax.experimental.pallas import tpu as pltpu
from jax.experimental.pallas import tpu_sc as plsc

N = 10000
E = 320000
R = 4
D = 128

NC = 2
NS = 16
NW = NC * NS

CHUNK = 96
CHUNKS_PER_TILE = 432
EP = NW * CHUNKS_PER_TILE * CHUNK
NACC = 10112
ROWS_PER_TILE = NACC // NS
IB = 24
NB = CHUNKS_PER_TILE // IB
DH = D // 2


def _mm_body(x_ref, w_ref, o_ref):
    y = jnp.dot(x_ref[...], w_ref[0],
                preferred_element_type=jnp.float32).astype(jnp.bfloat16)
    lo = jax.lax.bitcast_convert_type(y[:, :DH], jnp.uint16).astype(jnp.uint32)
    hi = jax.lax.bitcast_convert_type(y[:, DH:], jnp.uint16).astype(jnp.uint32)
    o_ref[0] = jax.lax.bitcast_convert_type(lo | (hi << 16), jnp.int32)


def _relation_matmuls(inp, weights):
    BN = 1000
    return pl.pallas_call(
        _mm_body,
        grid=(R, N // BN),
        in_specs=[
            pl.BlockSpec((BN, D), lambda r, i: (i, 0)),
            pl.BlockSpec((1, D, D), lambda r, i: (r, 0, 0)),
        ],
        out_specs=pl.BlockSpec((1, BN, DH), lambda r, i: (r, i, 0)),
        out_shape=jax.ShapeDtypeStruct((R, N, DH), jnp.int32),
    )(inp, weights)


def _sc_body(table_hbm, src_hbm, dst_hbm, out0_hbm, out1_hbm,
             src_a, src_b, dst_a, dst_b, g0, g1, f0, f1, acc_sh,
             sem_g0, sem_g1, sem_s0, sem_s1, sem_i0, sem_i1):
    c = lax.axis_index("c")
    s = lax.axis_index("s")
    wid = s * NC + c
    rows0 = f0

    def zrow(i, _):
        def zcol(j, _):
            rows0[i, pl.ds(j * 16, 16)] = jnp.zeros((16,), jnp.float32)
            return 0
        return lax.fori_loop(0, D // 16, zcol, 0)
    lax.fori_loop(0, CHUNK, zrow, 0)

    row0 = s * ROWS_PER_TILE
    for k in range(ROWS_PER_TILE // CHUNK):
        pltpu.sync_copy(rows0, acc_sh.at[pl.ds(row0 + k * CHUNK, CHUNK)])
    tail = ROWS_PER_TILE % CHUNK
    pltpu.sync_copy(rows0.at[pl.ds(0, tail)],
                    acc_sh.at[pl.ds(row0 + (ROWS_PER_TILE // CHUNK) * CHUNK, tail)])

    plsc.subcore_barrier()

    cbase = wid * CHUNKS_PER_TILE
    idx_bufs = [(src_a, dst_a, sem_i0), (src_b, dst_b, sem_i1)]
    ring = [(g0, f0, sem_g0, sem_s0), (g1, f1, sem_g1, sem_s1)]
    NBUF = len(ring)

    def convert(gbuf, fbuf):
        def conv_row(i, _):
            for g in range(DH // 16):
                w = gbuf[i, pl.ds(g * 16, 16)]
                a = jax.lax.bitcast_convert_type(jnp.left_shift(w, 16),
                                                 jnp.float32)
                b = jax.lax.bitcast_convert_type(w & jnp.int32(-65536),
                                                 jnp.float32)
                fbuf[i, pl.ds(g * 16, 16)] = a
                fbuf[i, pl.ds(DH + g * 16, 16)] = b
            return 0
        lax.fori_loop(0, CHUNK, conv_row, 0)

    def fetch_idx(p):
        sblk, dblk, sem = idx_bufs[p % 2]
        pltpu.async_copy(src_hbm.at[pl.ds(cbase + p * IB, IB)], sblk, sem)
        pltpu.async_copy(dst_hbm.at[pl.ds(cbase + p * IB, IB)], dblk, sem)

    def wait_idx(p):
        sblk, dblk, sem = idx_bufs[p % 2]
        pltpu.make_async_copy(src_hbm.at[pl.ds(0, IB)], sblk, sem).wait()
        pltpu.make_async_copy(dst_hbm.at[pl.ds(0, IB)], dblk, sem).wait()

    def wait_gather(buf, sem):
        pltpu.make_async_copy(table_hbm.at[pl.ds(0, CHUNK)], buf, sem).wait()

    fetch_idx(0)
    for p in range(NB):
        sblk, dblk, _ = idx_bufs[p % 2]
        if p + 1 < NB:
            fetch_idx(p + 1)
        wait_idx(p)

        for b, (gbuf, fbuf, sem_g, _) in enumerate(ring):
            pltpu.async_copy(table_hbm.at[sblk.at[b]], gbuf, sem_g)

        def grp(g, _, sblk=sblk, dblk=dblk):
            scats = []
            for b, (gbuf, fbuf, sem_g, sem_s) in enumerate(ring):
                j = NBUF * g + b
                wait_gather(gbuf, sem_g)
                convert(gbuf, fbuf)

                @pl.when(j + NBUF < IB)
                def _(gbuf=gbuf, sem_g=sem_g, j=j, sblk=sblk):
                    pltpu.async_copy(table_hbm.at[sblk.at[j + NBUF]], gbuf, sem_g)
                scats.append(
                    pltpu.async_copy(fbuf, acc_sh.at[dblk.at[j]], sem_s, add=True))
            for sc in scats:
                sc.wait()
            return 0
        lax.fori_loop(0, IB // NBUF, grp, 0)

    plsc.subcore_barrier()

    @pl.when(c == 0)
    def _():
        pltpu.sync_copy(acc_sh.at[pl.ds(row0, ROWS_PER_TILE)],
                        out0_hbm.at[pl.ds(row0, ROWS_PER_TILE)])

    @pl.when(c == 1)
    def _():
        pltpu.sync_copy(acc_sh.at[pl.ds(row0, ROWS_PER_TILE)],
                        out1_hbm.at[pl.ds(row0, ROWS_PER_TILE)])


_sc_spmm = functools.partial(
    pl.kernel,
    out_type=(
        jax.ShapeDtypeStruct((NACC, D), jnp.float32),
        jax.ShapeDtypeStruct((NACC, D), jnp.float32),
    ),
    mesh=plsc.VectorSubcoreMesh(core_axis_name="c", subcore_axis_name="s"),
    compiler_params=pltpu.CompilerParams(use_tc_tiling_on_sc=False),
    scratch_types=[
        pltpu.VMEM((IB, CHUNK), jnp.int32),
        pltpu.VMEM((IB, CHUNK), jnp.int32),
        pltpu.VMEM((IB, CHUNK), jnp.int32),
        pltpu.VMEM((IB, CHUNK), jnp.int32),
        pltpu.VMEM((CHUNK, DH), jnp.int32),
        pltpu.VMEM((CHUNK, DH), jnp.int32),
        pltpu.VMEM((CHUNK, D), jnp.float32),
        pltpu.VMEM((CHUNK, D), jnp.float32),
        pltpu.VMEM_SHARED((NACC, D), jnp.float32),
        pltpu.SemaphoreType.DMA,
        pltpu.SemaphoreType.DMA,
        pltpu.SemaphoreType.DMA,
        pltpu.SemaphoreType.DMA,
        pltpu.SemaphoreType.DMA,
        pltpu.SemaphoreType.DMA,
    ],
)(_sc_body)


def _combine_body(p0_ref, p1_ref, b_ref, o_ref):
    bias_sum = jnp.sum(b_ref[...], axis=0, keepdims=True)
    o_ref[...] = p0_ref[...] + p1_ref[...] + bias_sum


def _combine(p0, p1, bias):
    BN = 1000
    return pl.pallas_call(
        _combine_body,
        grid=(N // BN,),
        in_specs=[
            pl.BlockSpec((BN, D), lambda i: (i, 0)),
            pl.BlockSpec((BN, D), lambda i: (i, 0)),
            pl.BlockSpec((R, D), lambda i: (0, 0)),
        ],
        out_specs=pl.BlockSpec((BN, D), lambda i: (i, 0)),
        out_shape=jax.ShapeDtypeStruct((N, D), jnp.float32),
    )(p0, p1, bias)


@jax.jit
def kernel(inp, edge_index, weights, bias):
    table = _relation_matmuls(inp, weights).reshape(R * N, DH)

    roff = (jnp.arange(R, dtype=jnp.int32) * N)[:, None]
    src = (edge_index[:, 1, :] + roff).reshape(-1)
    dst = edge_index[:, 0, :].reshape(-1)
    npad = EP - R * E
    pad_i = jnp.arange(npad, dtype=jnp.int32)
    src = jnp.concatenate([src, pad_i % (R * N)])
    dst = jnp.concatenate([dst, N + pad_i % (NACC - N)])
    src = src.reshape(EP // CHUNK, CHUNK)
    dst = dst.reshape(EP // CHUNK, CHUNK)

    p0, p1 = _sc_spmm(table, src, dst)
    return _combine(p0, p1, bias)


# speedup vs baseline: 1.9660x; 1.9660x over previous
import functools

import jax
import jax.numpy as jnp
from jax import lax
from jax.experimental import pallas as pl
from jax.experimental.pallas import tpu as pltpu
from jax.experimental.pallas import tpu_sc as plsc

N = 10000
E = 320000
R = 4
D = 128

NC = 2
NS = 16
NW = NC * NS

CHUNK = 96
CHUNKS_PER_TILE = 432
EP = NW * CHUNKS_PER_TILE * CHUNK
NACC = 10112
ROWS_PER_TILE = NACC // NS
IB = 24
NB = CHUNKS_PER_TILE // IB
DH = D // 2


def _mm_body(x_ref, w_ref, o_ref):
    y = jnp.dot(x_ref[...], w_ref[0],
                preferred_element_type=jnp.float32).astype(jnp.bfloat16)
    lo = jax.lax.bitcast_convert_type(y[:, :DH], jnp.uint16).astype(jnp.uint32)
    hi = jax.lax.bitcast_convert_type(y[:, DH:], jnp.uint16).astype(jnp.uint32)
    o_ref[0] = jax.lax.bitcast_convert_type(lo | (hi << 16), jnp.int32)


def _relation_matmuls(inp, weights):
    BN = 1000
    return pl.pallas_call(
        _mm_body,
        grid=(R, N // BN),
        in_specs=[
            pl.BlockSpec((BN, D), lambda r, i: (i, 0)),
            pl.BlockSpec((1, D, D), lambda r, i: (r, 0, 0)),
        ],
        out_specs=pl.BlockSpec((1, BN, DH), lambda r, i: (r, i, 0)),
        out_shape=jax.ShapeDtypeStruct((R, N, DH), jnp.int32),
    )(inp, weights)


def _sc_body(table_hbm, src_hbm, dst_hbm, out0_hbm, out1_hbm,
             src_a, src_b, dst_a, dst_b, g0, g1, f0, f1, acc_sh,
             sem_g0, sem_g1, sem_s0, sem_s1, sem_i0, sem_i1):
    c = lax.axis_index("c")
    s = lax.axis_index("s")
    wid = s * NC + c
    rows0 = f0

    def zrow(i, _):
        def zcol(j, _):
            rows0[i, pl.ds(j * 16, 16)] = jnp.zeros((16,), jnp.float32)
            return 0
        return lax.fori_loop(0, D // 16, zcol, 0)
    lax.fori_loop(0, CHUNK, zrow, 0)

    row0 = s * ROWS_PER_TILE
    for k in range(ROWS_PER_TILE // CHUNK):
        pltpu.sync_copy(rows0, acc_sh.at[pl.ds(row0 + k * CHUNK, CHUNK)])
    tail = ROWS_PER_TILE % CHUNK
    pltpu.sync_copy(rows0.at[pl.ds(0, tail)],
                    acc_sh.at[pl.ds(row0 + (ROWS_PER_TILE // CHUNK) * CHUNK, tail)])

    plsc.subcore_barrier()

    cbase = wid * CHUNKS_PER_TILE
    idx_bufs = [(src_a, dst_a, sem_i0), (src_b, dst_b, sem_i1)]
    ring = [(g0, f0, sem_g0, sem_s0), (g1, f1, sem_g1, sem_s1)]
    NBUF = len(ring)

    def convert(gbuf, fbuf):
        @plsc.parallel_loop(0, CHUNK, unroll=8)
        def conv_row(i):
            for g in range(DH // 16):
                w = gbuf[i, pl.ds(g * 16, 16)]
                a = jax.lax.bitcast_convert_type(jnp.left_shift(w, 16),
                                                 jnp.float32)
                b = jax.lax.bitcast_convert_type(w & jnp.int32(-65536),
                                                 jnp.float32)
                fbuf[i, pl.ds(g * 16, 16)] = a
                fbuf[i, pl.ds(DH + g * 16, 16)] = b

    def fetch_idx(p):
        sblk, dblk, sem = idx_bufs[p % 2]
        pltpu.async_copy(src_hbm.at[pl.ds(cbase + p * IB, IB)], sblk, sem)
        pltpu.async_copy(dst_hbm.at[pl.ds(cbase + p * IB, IB)], dblk, sem)

    def wait_idx(p):
        sblk, dblk, sem = idx_bufs[p % 2]
        pltpu.make_async_copy(src_hbm.at[pl.ds(0, IB)], sblk, sem).wait()
        pltpu.make_async_copy(dst_hbm.at[pl.ds(0, IB)], dblk, sem).wait()

    def wait_gather(buf, sem):
        pltpu.make_async_copy(table_hbm.at[pl.ds(0, CHUNK)], buf, sem).wait()

    fetch_idx(0)
    for p in range(NB):
        sblk, dblk, _ = idx_bufs[p % 2]
        if p + 1 < NB:
            fetch_idx(p + 1)
        wait_idx(p)

        for b, (gbuf, fbuf, sem_g, _) in enumerate(ring):
            pltpu.async_copy(table_hbm.at[sblk.at[b]], gbuf, sem_g)

        def grp(g, _, sblk=sblk, dblk=dblk):
            scats = []
            for b, (gbuf, fbuf, sem_g, sem_s) in enumerate(ring):
                j = NBUF * g + b
                wait_gather(gbuf, sem_g)
                convert(gbuf, fbuf)

                @pl.when(j + NBUF < IB)
                def _(gbuf=gbuf, sem_g=sem_g, j=j, sblk=sblk):
                    pltpu.async_copy(table_hbm.at[sblk.at[j + NBUF]], gbuf, sem_g)
                scats.append(
                    pltpu.async_copy(fbuf, acc_sh.at[dblk.at[j]], sem_s, add=True))
            for sc in scats:
                sc.wait()
            return 0
        lax.fori_loop(0, IB // NBUF, grp, 0)

    plsc.subcore_barrier()

    @pl.when(c == 0)
    def _():
        pltpu.sync_copy(acc_sh.at[pl.ds(row0, ROWS_PER_TILE)],
                        out0_hbm.at[pl.ds(row0, ROWS_PER_TILE)])

    @pl.when(c == 1)
    def _():
        pltpu.sync_copy(acc_sh.at[pl.ds(row0, ROWS_PER_TILE)],
                        out1_hbm.at[pl.ds(row0, ROWS_PER_TILE)])


_sc_spmm = functools.partial(
    pl.kernel,
    out_type=(
        jax.ShapeDtypeStruct((NACC, D), jnp.float32),
        jax.ShapeDtypeStruct((NACC, D), jnp.float32),
    ),
    mesh=plsc.VectorSubcoreMesh(core_axis_name="c", subcore_axis_name="s"),
    compiler_params=pltpu.CompilerParams(use_tc_tiling_on_sc=False),
    scratch_types=[
        pltpu.VMEM((IB, CHUNK), jnp.int32),
        pltpu.VMEM((IB, CHUNK), jnp.int32),
        pltpu.VMEM((IB, CHUNK), jnp.int32),
        pltpu.VMEM((IB, CHUNK), jnp.int32),
        pltpu.VMEM((CHUNK, DH), jnp.int32),
        pltpu.VMEM((CHUNK, DH), jnp.int32),
        pltpu.VMEM((CHUNK, D), jnp.float32),
        pltpu.VMEM((CHUNK, D), jnp.float32),
        pltpu.VMEM_SHARED((NACC, D), jnp.float32),
        pltpu.SemaphoreType.DMA,
        pltpu.SemaphoreType.DMA,
        pltpu.SemaphoreType.DMA,
        pltpu.SemaphoreType.DMA,
        pltpu.SemaphoreType.DMA,
        pltpu.SemaphoreType.DMA,
    ],
)(_sc_body)


def _combine_body(p0_ref, p1_ref, b_ref, o_ref):
    bias_sum = jnp.sum(b_ref[...], axis=0, keepdims=True)
    o_ref[...] = p0_ref[...] + p1_ref[...] + bias_sum


def _combine(p0, p1, bias):
    BN = 1000
    return pl.pallas_call(
        _combine_body,
        grid=(N // BN,),
        in_specs=[
            pl.BlockSpec((BN, D), lambda i: (i, 0)),
            pl.BlockSpec((BN, D), lambda i: (i, 0)),
            pl.BlockSpec((R, D), lambda i: (0, 0)),
        ],
        out_specs=pl.BlockSpec((BN, D), lambda i: (i, 0)),
        out_shape=jax.ShapeDtypeStruct((N, D), jnp.float32),
    )(p0, p1, bias)


@jax.jit
def kernel(inp, edge_index, weights, bias):
    table = _relation_matmuls(inp, weights).reshape(R * N, DH)

    roff = (jnp.arange(R, dtype=jnp.int32) * N)[:, None]
    src = (edge_index[:, 1, :] + roff).reshape(-1)
    dst = edge_index[:, 0, :].reshape(-1)
    npad = EP - R * E
    pad_i = jnp.arange(npad, dtype=jnp.int32)
    src = jnp.concatenate([src, pad_i % (R * N)])
    dst = jnp.concatenate([dst, N + pad_i % (NACC - N)])
    src = src.reshape(EP // CHUNK, CHUNK)
    dst = dst.reshape(EP // CHUNK, CHUNK)

    p0, p1 = _sc_spmm(table, src, dst)
    return _combine(p0, p1, bias)
